# Initial kernel scaffold; baseline (speedup 1.0000x reference)
import jax
import jax.numpy as jnp
from jax.experimental import pallas as pl


def kernel(attention_mx, W, b):
    raise NotImplementedError("write your pallas kernel here")


# speedup vs baseline: 1.3453x; 1.3453x over previous
import functools

import jax
import jax.numpy as jnp
from jax.experimental import pallas as pl
from jax.experimental.pallas import tpu as pltpu

N = 10000
D = 512
H = 8
BLK = 1000


def _body(x_ref, w_ref, b_ref, o_ref, c_ref, m_ref, s_ref, acc_ref):
    i = pl.program_id(0)

    @pl.when(i == 0)
    def _init():
        x0 = x_ref[0:1, :]
        w1 = w_ref[:, :D]
        c_ref[...] = jax.lax.dot_general(
            x0, w1, (((1,), (1,)), ((), ())),
            preferred_element_type=jnp.float32) + b_ref[...]
        m_ref[...] = jnp.full_like(m_ref, -jnp.inf)
        s_ref[...] = jnp.zeros_like(s_ref)
        acc_ref[...] = jnp.zeros_like(acc_ref)

    x = x_ref[...]
    w2 = w_ref[:, D:]
    l = jax.lax.dot_general(
        x, w2, (((1,), (1,)), ((), ())),
        preferred_element_type=jnp.float32) + c_ref[...]
    l = jnp.where(l >= 0, l, 0.01 * l)

    m_old = m_ref[...]
    m_new = jnp.maximum(m_old, jnp.max(l, axis=0, keepdims=True))
    p = jnp.exp(l - m_new)
    r = jnp.exp(m_old - m_new)
    m_ref[...] = m_new
    s_ref[...] = s_ref[...] * r + jnp.sum(p, axis=0, keepdims=True)
    acc_ref[...] = acc_ref[...] * r.T + jax.lax.dot_general(
        p, x, (((0,), (0,)), ((), ())),
        preferred_element_type=jnp.float32)

    @pl.when(i == pl.num_programs(0) - 1)
    def _fin():
        head_avg = jnp.sum(acc_ref[...] / s_ref[...].T, axis=0,
                           keepdims=True) / H
        o_ref[...] = jnp.maximum(head_avg, 0.0)


@jax.jit
def _run(attention_mx, W, b):
    out = pl.pallas_call(
        _body,
        grid=(N // BLK,),
        in_specs=[
            pl.BlockSpec((BLK, D), lambda i: (i, 0)),
            pl.BlockSpec((H, 2 * D), lambda i: (0, 0)),
            pl.BlockSpec((1, H), lambda i: (0, 0)),
        ],
        out_specs=pl.BlockSpec((1, D), lambda i: (0, 0)),
        out_shape=jax.ShapeDtypeStruct((1, D), jnp.float32),
        scratch_shapes=[
            pltpu.VMEM((1, H), jnp.float32),
            pltpu.VMEM((1, H), jnp.float32),
            pltpu.VMEM((1, H), jnp.float32),
            pltpu.VMEM((H, D), jnp.float32),
        ],
    )(attention_mx, W, b.reshape(1, H))
    return out.reshape(D)


def kernel(attention_mx, W, b):
    return _run(attention_mx, W, b)


# speedup vs baseline: 1.5797x; 1.1743x over previous
import functools

import jax
import jax.numpy as jnp
from jax.experimental import pallas as pl
from jax.experimental.pallas import tpu as pltpu

N = 10000
D = 512
H = 8
BLK = 2000


def _body(x_ref, w_ref, b_ref, o_ref, c_ref, m_ref, s_ref, acc_ref):
    i = pl.program_id(0)

    @pl.when(i == 0)
    def _init():
        x0 = x_ref[0:1, :]
        w1 = w_ref[:, :D]
        c_ref[...] = jax.lax.dot_general(
            x0, w1, (((1,), (1,)), ((), ())),
            preferred_element_type=jnp.float32) + b_ref[...]
        m_ref[...] = jnp.full_like(m_ref, -jnp.inf)
        s_ref[...] = jnp.zeros_like(s_ref)
        acc_ref[...] = jnp.zeros_like(acc_ref)

    x = x_ref[...]
    w2 = w_ref[:, D:]
    l = jax.lax.dot_general(
        x, w2, (((1,), (1,)), ((), ())),
        preferred_element_type=jnp.float32) + c_ref[...]
    l = jnp.where(l >= 0, l, 0.01 * l)

    m_old = m_ref[...]
    m_new = jnp.maximum(m_old, jnp.max(l, axis=0, keepdims=True))
    p = jnp.exp(l - m_new)
    r = jnp.exp(m_old - m_new)
    m_ref[...] = m_new
    s_ref[...] = s_ref[...] * r + jnp.sum(p, axis=0, keepdims=True)
    acc_ref[...] = acc_ref[...] * r.T + jax.lax.dot_general(
        p, x, (((0,), (0,)), ((), ())),
        preferred_element_type=jnp.float32)

    @pl.when(i == pl.num_programs(0) - 1)
    def _fin():
        head_avg = jnp.sum(acc_ref[...] / s_ref[...].T, axis=0,
                           keepdims=True) / H
        o_ref[...] = jnp.maximum(head_avg, 0.0)


@jax.jit
def _run(attention_mx, W, b):
    out = pl.pallas_call(
        _body,
        grid=(N // BLK,),
        in_specs=[
            pl.BlockSpec((BLK, D), lambda i: (i, 0)),
            pl.BlockSpec((H, 2 * D), lambda i: (0, 0)),
            pl.BlockSpec((1, H), lambda i: (0, 0)),
        ],
        out_specs=pl.BlockSpec((1, D), lambda i: (0, 0)),
        out_shape=jax.ShapeDtypeStruct((1, D), jnp.float32),
        scratch_shapes=[
            pltpu.VMEM((1, H), jnp.float32),
            pltpu.VMEM((1, H), jnp.float32),
            pltpu.VMEM((1, H), jnp.float32),
            pltpu.VMEM((H, D), jnp.float32),
        ],
    )(attention_mx, W, b.reshape(1, H))
    return out.reshape(D)


def kernel(attention_mx, W, b):
    return _run(attention_mx, W, b)
